# baseline (speedup 1.0000x reference)
import functools

import jax
import jax.numpy as jnp
from jax import lax
from jax.experimental import pallas as pl
from jax.experimental.pallas import tpu as pltpu
from jax.experimental.pallas import tpu_sc as plsc

N = 10000
D = 128
NC = 2
NS = 16
NW = NC * NS
K = 128
NB = 80
EPT = NB * K
E_PAD = NW * EPT
NP = 10240
RPT = NP // NS
BLK = 512
GRID = NP // BLK
DEG_W = 128


def _mesh():
    return plsc.VectorSubcoreMesh(core_axis_name="c", subcore_axis_name="s")


KS = 32
NBS = EPT // KS
ICH = 8
NCH = NBS // ICH
NIB = 4
SLOTS = 10
LAG = 3


def _sc_scatter_rows(y, eidx, zeros):
    @functools.partial(
        pl.kernel,
        out_type=jax.ShapeDtypeStruct((NC, NP, D), jnp.float32),
        mesh=_mesh(),
        scratch_types=[
            pltpu.VMEM_SHARED((NP, D), jnp.float32),
            pltpu.VMEM((NIB, ICH, 2, KS), jnp.int32),
            pltpu.VMEM((SLOTS, KS, D), jnp.float32),
        ] + [pltpu.SemaphoreType.DMA] * (2 * SLOTS + 2),
    )
    def k(y_hbm, e_hbm, z_hbm, out_hbm, acc, ich, rows, *sems):
        c = lax.axis_index("c")
        s = lax.axis_index("s")
        wid = c * NS + s
        r0 = s * RPT
        gsem = sems[:SLOTS]
        ssem = sems[SLOTS:2 * SLOTS]
        isem = sems[2 * SLOTS:]

        pltpu.sync_copy(z_hbm.at[pl.ds(r0, RPT)], acc.at[pl.ds(r0, RPT)])
        pltpu.sync_copy(e_hbm.at[wid, 0], ich.at[0])
        pltpu.async_copy(e_hbm.at[wid, 1], ich.at[1], isem[1])
        plsc.subcore_barrier()

        def start_gather(i, b):
            pltpu.async_copy(
                y_hbm.at[ich.at[(i // ICH) % NIB, i % ICH, 0]],
                rows.at[b], gsem[b])

        def wait_gather(b):
            pltpu.make_async_copy(y_hbm.at[ich.at[0, 0, 0]], rows.at[b],
                                  gsem[b]).wait()

        def start_scat(i, b):
            pltpu.async_copy(
                rows.at[b], acc.at[ich.at[(i // ICH) % NIB, i % ICH, 1]],
                ssem[b], add=True)

        def wait_scat(b):
            pltpu.make_async_copy(rows.at[b], acc.at[ich.at[0, 0, 1]],
                                  ssem[b]).wait()

        for b in range(SLOTS):
            if b == ICH:
                pltpu.make_async_copy(e_hbm.at[wid, 1], ich.at[1],
                                      isem[1]).wait()
            start_gather(b, b)
        for i in range(NBS):
            b = i % SLOTS
            ci = i // ICH
            if i % ICH == 0 and ci + 2 < NCH:
                pltpu.async_copy(e_hbm.at[wid, ci + 2],
                                 ich.at[(ci + 2) % NIB], isem[ci % 2])
            if i >= LAG:
                wait_scat((i - LAG) % SLOTS)
                j = i - LAG + SLOTS
                if j < NBS:
                    if j % ICH == 0:
                        cj = j // ICH
                        pltpu.make_async_copy(e_hbm.at[wid, 0], ich.at[0],
                                              isem[cj % 2]).wait()
                    start_gather(j, (i - LAG) % SLOTS)
            wait_gather(b)
            start_scat(i, b)
        for t in range(LAG):
            wait_scat((NBS - LAG + t) % SLOTS)

        plsc.subcore_barrier()
        pltpu.sync_copy(acc.at[pl.ds(r0, RPT)], out_hbm.at[c, pl.ds(r0, RPT)])

    return k(y, eidx, zeros)


def _sc_degree(dsts, ones, zeros16):
    CH = 16

    @functools.partial(
        pl.kernel,
        out_type=jax.ShapeDtypeStruct((NC, NP, DEG_W), jnp.float32),
        mesh=_mesh(),
        scratch_types=[
            pltpu.VMEM_SHARED((NP, DEG_W), jnp.float32),
            pltpu.VMEM((NB, K), jnp.int32),
            pltpu.VMEM((K, DEG_W), jnp.float32),
            pltpu.SemaphoreType.DMA,
        ],
    )
    def k(dst_hbm, ones_hbm, z_hbm, out_hbm, acc, idxd, ones_v, sem):
        c = lax.axis_index("c")
        s = lax.axis_index("s")
        wid = c * NS + s
        r0 = s * RPT

        pltpu.sync_copy(z_hbm.at[pl.ds(r0, RPT)], acc.at[pl.ds(r0, RPT)])
        pltpu.sync_copy(dst_hbm.at[wid], idxd)
        pltpu.sync_copy(ones_hbm, ones_v)
        plsc.subcore_barrier()

        @pl.loop(0, NB, step=CH)
        def _(i0):
            for j in range(CH):
                pltpu.async_copy(ones_v, acc.at[idxd.at[i0 + j]], sem,
                                 add=True)
            for j in range(CH):
                pltpu.make_async_copy(ones_v, acc.at[idxd.at[0]],
                                      sem).wait()

        plsc.subcore_barrier()
        pltpu.sync_copy(acc.at[pl.ds(r0, RPT)], out_hbm.at[c, pl.ds(r0, RPT)])

    return k(dsts, ones, zeros16)


def _row_spec(w=D):
    return pl.BlockSpec((BLK, w), lambda i: (i, 0))


def _full_spec(shape):
    return pl.BlockSpec(shape, lambda i: (0, 0))


def _dinv_body(d0_ref, d1_ref, o_ref):
    deg = d0_ref[:, 0:1] + d1_ref[:, 0:1] + 1.0
    o_ref[...] = jnp.broadcast_to(lax.rsqrt(jnp.maximum(deg, 1.0)),
                                  o_ref.shape)


def _tc_dinv(d0, d1):
    return pl.pallas_call(
        _dinv_body,
        grid=(GRID,),
        in_specs=[_row_spec(DEG_W), _row_spec(DEG_W)],
        out_specs=_row_spec(),
        out_shape=jax.ShapeDtypeStruct((NP, D), jnp.float32),
    )(d0, d1)


def _prep_body(x_ref, w_ref, dv_ref, y_ref):
    y_ref[...] = dv_ref[...] * jnp.dot(x_ref[...], w_ref[...],
                                       preferred_element_type=jnp.float32)


def _tc_prep(x, w, dv):
    return pl.pallas_call(
        _prep_body,
        grid=(GRID,),
        in_specs=[_row_spec(), _full_spec((D, D)), _row_spec()],
        out_specs=_row_spec(),
        out_shape=jax.ShapeDtypeStruct((NP, D), jnp.float32),
    )(x, w, dv)


def _epi_body(p0_ref, p1_ref, y_ref, dv_ref, b_ref, w_ref, o_ref):
    dinv = dv_ref[...]
    h = jnp.maximum(dinv * (p0_ref[...] + p1_ref[...] + y_ref[...])
                    + b_ref[...], 0.0)
    o_ref[...] = dinv * jnp.dot(h, w_ref[...],
                                preferred_element_type=jnp.float32)


def _tc_epilogue(p0, p1, y, dv, b, w_next):
    return pl.pallas_call(
        _epi_body,
        grid=(GRID,),
        in_specs=[_row_spec(), _row_spec(), _row_spec(), _row_spec(),
                  _full_spec((1, D)), _full_spec((D, D))],
        out_specs=_row_spec(),
        out_shape=jax.ShapeDtypeStruct((NP, D), jnp.float32),
    )(p0, p1, y, dv, b.reshape(1, D), w_next)


def _fin_body(p0_ref, p1_ref, y_ref, dv_ref, b_ref, wl_ref, bl_ref, o_ref):
    dinv = dv_ref[...]
    h = jnp.maximum(dinv * (p0_ref[...] + p1_ref[...] + y_ref[...])
                    + b_ref[...], 0.0)
    o_ref[...] = jnp.maximum(
        jnp.dot(h, wl_ref[...], preferred_element_type=jnp.float32)
        + bl_ref[...], 0.0)


def _tc_final(p0, p1, y, dv, b, wl, bl):
    return pl.pallas_call(
        _fin_body,
        grid=(GRID,),
        in_specs=[_row_spec(), _row_spec(), _row_spec(), _row_spec(),
                  _full_spec((1, D)), _full_spec((D, D)), _full_spec((1, D))],
        out_specs=_row_spec(),
        out_shape=jax.ShapeDtypeStruct((NP, D), jnp.float32),
    )(p0, p1, y, dv, b.reshape(1, D), wl, bl.reshape(1, D))


def kernel(x, edge_index, edge_attr, W1, b1, W2, b2, W3, b3, W4, b4, W5, b5,
           Wl, bl):
    del edge_attr
    n = x.shape[0]
    e = edge_index.shape[1]
    npad = E_PAD - e
    j = jnp.arange(npad, dtype=jnp.int32)
    pad_src = (j * 131) % n
    pad_dst = n + (j % (NP - n))
    srcs = jnp.concatenate([edge_index[0].astype(jnp.int32),
                            pad_src]).reshape(NW, NB, K)
    dsts = jnp.concatenate([edge_index[1].astype(jnp.int32),
                            pad_dst]).reshape(NW, NB, K)
    eidx = jnp.stack([srcs.reshape(NW, NBS, KS),
                      dsts.reshape(NW, NBS, KS)], axis=2)
    eidx = eidx.reshape(NW, NCH, ICH, 2, KS)

    xp = jnp.zeros((NP, D), jnp.float32).at[:n].set(x)
    zeros = jnp.zeros((NP, D), jnp.float32)
    zeros16 = jnp.zeros((NP, DEG_W), jnp.float32)
    ones = jnp.ones((K, DEG_W), jnp.float32)

    degp = _sc_degree(dsts, ones, zeros16)
    dv = _tc_dinv(degp[0], degp[1])
    y = _tc_prep(xp, W1, dv)

    for b_l, w_next in ((b1, W2), (b2, W3), (b3, W4), (b4, W5)):
        p = _sc_scatter_rows(y, eidx, zeros)
        y = _tc_epilogue(p[0], p[1], y, dv, b_l, w_next)

    p = _sc_scatter_rows(y, eidx, zeros)
    out = _tc_final(p[0], p[1], y, dv, b5, Wl, bl)
    return out[:n]


# speedup vs baseline: 1.0468x; 1.0468x over previous
import functools

import jax
import jax.numpy as jnp
from jax import lax
from jax.experimental import pallas as pl
from jax.experimental.pallas import tpu as pltpu
from jax.experimental.pallas import tpu_sc as plsc

N = 10000
D = 128
NC = 2
NS = 16
NW = NC * NS
K = 128
NB = 80
EPT = NB * K
E_PAD = NW * EPT
NP = 10240
RPT = NP // NS
BLK = 512
GRID = NP // BLK
DEG_W = 128


def _mesh():
    return plsc.VectorSubcoreMesh(core_axis_name="c", subcore_axis_name="s")


KS = 64
NBS = EPT // KS
ICH = 8
NCH = NBS // ICH
NIB = 4
SLOTS = 5
LAG = 1


def _sc_scatter_rows(y, eidx, zeros):
    @functools.partial(
        pl.kernel,
        out_type=jax.ShapeDtypeStruct((NC, NP, D), jnp.float32),
        mesh=_mesh(),
        scratch_types=[
            pltpu.VMEM_SHARED((NP, D), jnp.float32),
            pltpu.VMEM((NIB, ICH, 2, KS), jnp.int32),
            pltpu.VMEM((SLOTS, KS, D), jnp.float32),
            pltpu.SemaphoreType.DMA,
            pltpu.SemaphoreType.DMA,
            pltpu.SemaphoreType.DMA,
            pltpu.SemaphoreType.DMA,
            pltpu.SemaphoreType.DMA,
            pltpu.SemaphoreType.DMA,
            pltpu.SemaphoreType.DMA,
            pltpu.SemaphoreType.DMA,
            pltpu.SemaphoreType.DMA,
            pltpu.SemaphoreType.DMA,
            pltpu.SemaphoreType.DMA,
            pltpu.SemaphoreType.DMA,
        ],
    )
    def k(y_hbm, e_hbm, z_hbm, out_hbm, acc, ich, rows,
          g0, g1, g2, g3, g4, s0, s1, s2, s3, s4, i0s, i1s):
        c = lax.axis_index("c")
        s = lax.axis_index("s")
        wid = c * NS + s
        r0 = s * RPT
        gsem = (g0, g1, g2, g3, g4)
        ssem = (s0, s1, s2, s3, s4)
        isem = (i0s, i1s)

        pltpu.sync_copy(z_hbm.at[pl.ds(r0, RPT)], acc.at[pl.ds(r0, RPT)])
        pltpu.sync_copy(e_hbm.at[wid, 0], ich.at[0])
        pltpu.async_copy(e_hbm.at[wid, 1], ich.at[1], isem[1])
        plsc.subcore_barrier()

        def start_gather(i, b):
            pltpu.async_copy(
                y_hbm.at[ich.at[(i // ICH) % NIB, i % ICH, 0]],
                rows.at[b], gsem[b])

        def wait_gather(b):
            pltpu.make_async_copy(y_hbm.at[ich.at[0, 0, 0]], rows.at[b],
                                  gsem[b]).wait()

        def start_scat(i, b):
            pltpu.async_copy(
                rows.at[b], acc.at[ich.at[(i // ICH) % NIB, i % ICH, 1]],
                ssem[b], add=True)

        def wait_scat(b):
            pltpu.make_async_copy(rows.at[b], acc.at[ich.at[0, 0, 1]],
                                  ssem[b]).wait()

        for b in range(SLOTS):
            start_gather(b, b)
        for i in range(NBS):
            b = i % SLOTS
            ci = i // ICH
            if i % ICH == 0 and ci + 2 < NCH:
                pltpu.async_copy(e_hbm.at[wid, ci + 2],
                                 ich.at[(ci + 2) % NIB], isem[ci % 2])
            if i >= LAG:
                wait_scat((i - LAG) % SLOTS)
                j = i - LAG + SLOTS
                if j < NBS:
                    if j % ICH == 0:
                        cj = j // ICH
                        pltpu.make_async_copy(e_hbm.at[wid, 0], ich.at[0],
                                              isem[cj % 2]).wait()
                    start_gather(j, (i - LAG) % SLOTS)
            wait_gather(b)
            start_scat(i, b)
        for t in range(LAG):
            wait_scat((NBS - LAG + t) % SLOTS)

        plsc.subcore_barrier()
        pltpu.sync_copy(acc.at[pl.ds(r0, RPT)], out_hbm.at[c, pl.ds(r0, RPT)])

    return k(y, eidx, zeros)


def _sc_degree(dsts, ones, zeros16):
    CH = 16

    @functools.partial(
        pl.kernel,
        out_type=jax.ShapeDtypeStruct((NC, NP, DEG_W), jnp.float32),
        mesh=_mesh(),
        scratch_types=[
            pltpu.VMEM_SHARED((NP, DEG_W), jnp.float32),
            pltpu.VMEM((NB, K), jnp.int32),
            pltpu.VMEM((K, DEG_W), jnp.float32),
            pltpu.SemaphoreType.DMA,
        ],
    )
    def k(dst_hbm, ones_hbm, z_hbm, out_hbm, acc, idxd, ones_v, sem):
        c = lax.axis_index("c")
        s = lax.axis_index("s")
        wid = c * NS + s
        r0 = s * RPT

        pltpu.sync_copy(z_hbm.at[pl.ds(r0, RPT)], acc.at[pl.ds(r0, RPT)])
        pltpu.sync_copy(dst_hbm.at[wid], idxd)
        pltpu.sync_copy(ones_hbm, ones_v)
        plsc.subcore_barrier()

        @pl.loop(0, NB, step=CH)
        def _(i0):
            for j in range(CH):
                pltpu.async_copy(ones_v, acc.at[idxd.at[i0 + j]], sem,
                                 add=True)
            for j in range(CH):
                pltpu.make_async_copy(ones_v, acc.at[idxd.at[0]],
                                      sem).wait()

        plsc.subcore_barrier()
        pltpu.sync_copy(acc.at[pl.ds(r0, RPT)], out_hbm.at[c, pl.ds(r0, RPT)])

    return k(dsts, ones, zeros16)


def _row_spec(w=D):
    return pl.BlockSpec((BLK, w), lambda i: (i, 0))


def _full_spec(shape):
    return pl.BlockSpec(shape, lambda i: (0, 0))


def _dinv_body(d0_ref, d1_ref, o_ref):
    deg = d0_ref[:, 0:1] + d1_ref[:, 0:1] + 1.0
    o_ref[...] = jnp.broadcast_to(lax.rsqrt(jnp.maximum(deg, 1.0)),
                                  o_ref.shape)


def _tc_dinv(d0, d1):
    return pl.pallas_call(
        _dinv_body,
        grid=(GRID,),
        in_specs=[_row_spec(DEG_W), _row_spec(DEG_W)],
        out_specs=_row_spec(),
        out_shape=jax.ShapeDtypeStruct((NP, D), jnp.float32),
    )(d0, d1)


def _prep_body(x_ref, w_ref, dv_ref, y_ref):
    y_ref[...] = dv_ref[...] * jnp.dot(x_ref[...], w_ref[...],
                                       preferred_element_type=jnp.float32)


def _tc_prep(x, w, dv):
    return pl.pallas_call(
        _prep_body,
        grid=(GRID,),
        in_specs=[_row_spec(), _full_spec((D, D)), _row_spec()],
        out_specs=_row_spec(),
        out_shape=jax.ShapeDtypeStruct((NP, D), jnp.float32),
    )(x, w, dv)


def _epi_body(p0_ref, p1_ref, y_ref, dv_ref, b_ref, w_ref, o_ref):
    dinv = dv_ref[...]
    h = jnp.maximum(dinv * (p0_ref[...] + p1_ref[...] + y_ref[...])
                    + b_ref[...], 0.0)
    o_ref[...] = dinv * jnp.dot(h, w_ref[...],
                                preferred_element_type=jnp.float32)


def _tc_epilogue(p0, p1, y, dv, b, w_next):
    return pl.pallas_call(
        _epi_body,
        grid=(GRID,),
        in_specs=[_row_spec(), _row_spec(), _row_spec(), _row_spec(),
                  _full_spec((1, D)), _full_spec((D, D))],
        out_specs=_row_spec(),
        out_shape=jax.ShapeDtypeStruct((NP, D), jnp.float32),
    )(p0, p1, y, dv, b.reshape(1, D), w_next)


def _fin_body(p0_ref, p1_ref, y_ref, dv_ref, b_ref, wl_ref, bl_ref, o_ref):
    dinv = dv_ref[...]
    h = jnp.maximum(dinv * (p0_ref[...] + p1_ref[...] + y_ref[...])
                    + b_ref[...], 0.0)
    o_ref[...] = jnp.maximum(
        jnp.dot(h, wl_ref[...], preferred_element_type=jnp.float32)
        + bl_ref[...], 0.0)


def _tc_final(p0, p1, y, dv, b, wl, bl):
    return pl.pallas_call(
        _fin_body,
        grid=(GRID,),
        in_specs=[_row_spec(), _row_spec(), _row_spec(), _row_spec(),
                  _full_spec((1, D)), _full_spec((D, D)), _full_spec((1, D))],
        out_specs=_row_spec(),
        out_shape=jax.ShapeDtypeStruct((NP, D), jnp.float32),
    )(p0, p1, y, dv, b.reshape(1, D), wl, bl.reshape(1, D))


def kernel(x, edge_index, edge_attr, W1, b1, W2, b2, W3, b3, W4, b4, W5, b5,
           Wl, bl):
    del edge_attr
    n = x.shape[0]
    e = edge_index.shape[1]
    npad = E_PAD - e
    j = jnp.arange(npad, dtype=jnp.int32)
    pad_src = (j * 131) % n
    pad_dst = n + (j % (NP - n))
    srcs = jnp.concatenate([edge_index[0].astype(jnp.int32),
                            pad_src]).reshape(NW, NB, K)
    dsts = jnp.concatenate([edge_index[1].astype(jnp.int32),
                            pad_dst]).reshape(NW, NB, K)
    eidx = jnp.stack([srcs.reshape(NW, NBS, KS),
                      dsts.reshape(NW, NBS, KS)], axis=2)
    eidx = eidx.reshape(NW, NCH, ICH, 2, KS)

    xp = jnp.zeros((NP, D), jnp.float32).at[:n].set(x)
    zeros = jnp.zeros((NP, D), jnp.float32)
    zeros16 = jnp.zeros((NP, DEG_W), jnp.float32)
    ones = jnp.ones((K, DEG_W), jnp.float32)

    degp = _sc_degree(dsts, ones, zeros16)
    dv = _tc_dinv(degp[0], degp[1])
    y = _tc_prep(xp, W1, dv)

    for b_l, w_next in ((b1, W2), (b2, W3), (b3, W4), (b4, W5)):
        p = _sc_scatter_rows(y, eidx, zeros)
        y = _tc_epilogue(p[0], p[1], y, dv, b_l, w_next)

    p = _sc_scatter_rows(y, eidx, zeros)
    out = _tc_final(p[0], p[1], y, dv, b5, Wl, bl)
    return out[:n]
